# baseline (speedup 1.0000x reference)
import functools

import jax
import jax.numpy as jnp
from jax import lax
from jax.experimental import pallas as pl
from jax.experimental.pallas import tpu as pltpu
from jax.experimental.pallas import tpu_sc as plsc

N1, N2, N3 = 4096, 8192, 2048
D = 128
HID = 64
NG = 32
DEPTH = 4

F = 144
CAP = 81920
CH = 128
NCH = 20
NSUB = 16
NCORE = 2

NP1, NP2, NP3 = N1 + 128, N2 + 128, N3 + 128


def _extract_edges(b, nr, nc):
    blke = 128
    nb = (nr * nc) // blke
    mf = b.reshape(nb, blke)
    counts = jnp.sum(mf, axis=1).astype(jnp.int32)
    offs = jnp.concatenate([jnp.zeros((1,), jnp.int32),
                            jnp.cumsum(counts)])
    nnz = offs[-1]
    blk = jnp.repeat(jnp.arange(nb, dtype=jnp.int32), counts,
                     total_repeat_length=CAP)
    rank = jnp.arange(CAP, dtype=jnp.int32) - offs[blk]
    rows = mf[blk]
    cums = jnp.cumsum(rows, axis=1)
    sel = (cums == (rank[:, None] + 1).astype(jnp.float32)) & (rows > 0)
    col_in_blk = jnp.argmax(sel, axis=1).astype(jnp.int32)
    flat = blk * blke + col_in_blk
    valid = jnp.arange(CAP, dtype=jnp.int32) < nnz
    r = jnp.where(valid, flat // nc, nr).astype(jnp.int32)
    c = jnp.where(valid, flat % nc, nc).astype(jnp.int32)
    shp = (NCORE * NSUB, NCH, CH)
    return (c.reshape(shp), r.reshape(shp),
            r.reshape(shp), c.reshape(shp))


def _sc_pass(table, src_idx, dst_idx, zeros_np, np_out):
    mesh = plsc.VectorSubcoreMesh(core_axis_name="c", subcore_axis_name="s")
    rp = np_out // NSUB

    @functools.partial(
        pl.kernel,
        out_type=jax.ShapeDtypeStruct((NCORE, np_out, F), jnp.float32),
        mesh=mesh,
        compiler_params=pltpu.CompilerParams(use_tc_tiling_on_sc=False),
        scratch_types=[
            pltpu.VMEM((NCH, CH), jnp.int32),
            pltpu.VMEM((NCH, CH), jnp.int32),
            pltpu.VMEM((CH, F), jnp.float32),
            pltpu.VMEM_SHARED((np_out, F), jnp.float32),
            pltpu.SemaphoreType.DMA,
        ],
    )
    def k(table_h, src_h, dst_h, zeros_h, out_h, src_v, dst_v, rows_v, acc_s, sem):
        core = lax.axis_index("c")
        sub = lax.axis_index("s")
        w = core * NSUB + sub
        pltpu.sync_copy(zeros_h.at[pl.ds(sub * rp, rp)],
                        acc_s.at[pl.ds(sub * rp, rp)])
        pltpu.sync_copy(src_h.at[w], src_v)
        pltpu.sync_copy(dst_h.at[w], dst_v)
        plsc.subcore_barrier()
        for j in range(NCH):
            pltpu.async_copy(table_h.at[src_v.at[j]], rows_v, sem).wait()
            pltpu.sync_copy(rows_v, acc_s.at[dst_v.at[j]], add=True)
        plsc.subcore_barrier()
        pltpu.sync_copy(acc_s.at[pl.ds(sub * rp, rp)],
                        out_h.at[core, pl.ds(sub * rp, rp)])

    return k(table, src_idx, dst_idx, zeros_np)


def _table(h, x, npad):
    n = h.shape[0]
    t = jnp.zeros((npad, F), jnp.float32)
    t = t.at[:n, :D].set(h)
    t = t.at[:n, D:D + 3].set(x)
    t = t.at[:n, D + 3].set(jnp.sum(x * x, axis=1))
    t = t.at[:n, D + 4].set(1.0)
    return t


def _affine(h, W, b):
    n = h.shape[0]
    blk = 1024

    def body(h_ref, w_ref, b_ref, o_ref):
        o_ref[...] = h_ref[...] @ w_ref[...] + b_ref[...]

    return pl.pallas_call(
        body,
        grid=(n // blk,),
        in_specs=[pl.BlockSpec((blk, D), lambda i: (i, 0)),
                  pl.BlockSpec((D, D), lambda i: (0, 0)),
                  pl.BlockSpec((1, D), lambda i: (0, 0))],
        out_specs=pl.BlockSpec((blk, D), lambda i: (i, 0)),
        out_shape=jax.ShapeDtypeStruct((n, D), jnp.float32),
    )(h, W, b.reshape(1, D))


def _lvl1_call(h, S, deg, d, ws):
    n = h.shape[0]
    blk = 1024

    def body(h_ref, s_ref, deg_ref, d_ref,
             uw1h, uw1a, uw1d, ub1, uw2, ub2,
             hw1h, hw1m, hb1, hw2, hb2, xw, xb,
             oh_ref, oc_ref):
        h_ = h_ref[...]
        aggh = s_ref[...] / deg_ref[...]
        hidu = jnp.maximum(h_ @ uw1h[...] + aggh @ uw1a[...]
                           + d_ref[...] * uw1d[...] + ub1[...], 0.0)
        m = hidu @ uw2[...] + ub2[...]
        hid2 = jnp.maximum(h_ @ hw1h[...] + m @ hw1m[...] + hb1[...], 0.0)
        oh_ref[...] = h_ + hid2 @ hw2[...] + hb2[...]
        oc_ref[...] = m @ xw[...] + xb[...]

    rspec = pl.BlockSpec((blk, D), lambda i: (i, 0))
    cspec = pl.BlockSpec((blk, 1), lambda i: (i, 0))

    def wspec(s):
        return pl.BlockSpec(s, lambda i: tuple(0 for _ in s))

    uW1, ub1, uW2, ub2, hW1, hb1, hW2, hb2, xW, xb = ws
    return pl.pallas_call(
        body,
        grid=(n // blk,),
        in_specs=[rspec, rspec, cspec, cspec,
                  wspec((D, HID)), wspec((D, HID)), wspec((1, HID)),
                  wspec((1, HID)), wspec((HID, D)), wspec((1, D)),
                  wspec((D, HID)), wspec((D, HID)), wspec((1, HID)),
                  wspec((HID, D)), wspec((1, D)),
                  wspec((D, 1)), wspec((1, 1))],
        out_specs=[rspec, cspec],
        out_shape=[jax.ShapeDtypeStruct((n, D), jnp.float32),
                   jax.ShapeDtypeStruct((n, 1), jnp.float32)],
    )(h, S, deg, d,
      uW1[:D], uW1[D:2 * D], uW1[2 * D:2 * D + 1], ub1.reshape(1, HID),
      uW2, ub2.reshape(1, D),
      hW1[:D], hW1[D:2 * D], hb1.reshape(1, HID), hW2, hb2.reshape(1, D),
      xW, xb.reshape(1, 1))


def _lvl2_call(h, Su, degu, du, Sd, degd, dd, ws):
    n = h.shape[0]
    blk = 1024

    def body(h_ref, su_ref, degu_ref, du_ref, sd_ref, degd_ref, dd_ref,
             uw1h, uw1a, uw1d, ub1, uw2, ub2,
             dw1h, dw1a, dw1d, db1, dw2, db2,
             hw1h, hw1u, hw1d, hb1, hw2, hb2,
             oh_ref):
        h_ = h_ref[...]
        agu = su_ref[...] / degu_ref[...]
        hid_u = jnp.maximum(h_ @ uw1h[...] + agu @ uw1a[...]
                            + du_ref[...] * uw1d[...] + ub1[...], 0.0)
        mu = hid_u @ uw2[...] + ub2[...]
        agd = sd_ref[...] / degd_ref[...]
        hid_d = jnp.maximum(h_ @ dw1h[...] + agd @ dw1a[...]
                            + dd_ref[...] * dw1d[...] + db1[...], 0.0)
        md = hid_d @ dw2[...] + db2[...]
        hid2 = jnp.maximum(h_ @ hw1h[...] + mu @ hw1u[...]
                           + md @ hw1d[...] + hb1[...], 0.0)
        oh_ref[...] = h_ + hid2 @ hw2[...] + hb2[...]

    rspec = pl.BlockSpec((blk, D), lambda i: (i, 0))
    cspec = pl.BlockSpec((blk, 1), lambda i: (i, 0))

    def wspec(s):
        return pl.BlockSpec(s, lambda i: tuple(0 for _ in s))

    uW1, ub1, uW2, ub2, dW1, db1, dW2, db2, hW1, hb1, hW2, hb2 = ws
    return pl.pallas_call(
        body,
        grid=(n // blk,),
        in_specs=[rspec, rspec, cspec, cspec, rspec, cspec, cspec,
                  wspec((D, HID)), wspec((D, HID)), wspec((1, HID)),
                  wspec((1, HID)), wspec((HID, D)), wspec((1, D)),
                  wspec((D, HID)), wspec((D, HID)), wspec((1, HID)),
                  wspec((1, HID)), wspec((HID, D)), wspec((1, D)),
                  wspec((D, HID)), wspec((D, HID)), wspec((D, HID)),
                  wspec((1, HID)), wspec((HID, D)), wspec((1, D))],
        out_specs=rspec,
        out_shape=jax.ShapeDtypeStruct((n, D), jnp.float32),
    )(h, Su, degu, du, Sd, degd, dd,
      uW1[:D], uW1[D:2 * D], uW1[2 * D:2 * D + 1], ub1.reshape(1, HID),
      uW2, ub2.reshape(1, D),
      dW1[:D], dW1[D:2 * D], dW1[2 * D:2 * D + 1], db1.reshape(1, HID),
      dW2, db2.reshape(1, D),
      hW1[:D], hW1[D:2 * D], hW1[2 * D:3 * D], hb1.reshape(1, HID),
      hW2, hb2.reshape(1, D))


def _segmax(hf, ids):
    n = hf.shape[0]

    def body(h_ref, id_ref, o_ref):
        h_ = h_ref[...]
        ids_ = id_ref[...]
        for g in range(NG):
            mk = ids_ == g
            row = jnp.max(jnp.where(mk, h_, -jnp.inf), axis=0, keepdims=True)
            o_ref[pl.ds(g, 1), :] = jnp.where(jnp.isfinite(row), row, 0.0)

    return pl.pallas_call(
        body,
        in_specs=[pl.BlockSpec((n, D), lambda: (0, 0)),
                  pl.BlockSpec((n, 1), lambda: (0, 0))],
        out_specs=pl.BlockSpec((NG, D), lambda: (0, 0)),
        out_shape=jax.ShapeDtypeStruct((NG, D), jnp.float32),
    )(hf, ids.reshape(n, 1).astype(jnp.int32))


def _classifier(H, cw, cb):
    W2p = jnp.zeros((256, 128), jnp.float32).at[:, :10].set(cw[2])
    b2p = jnp.zeros((1, 128), jnp.float32).at[0, :10].set(cb[2])

    def body(h_ref, w0, b0, w1, b1, w2, b2, o_ref):
        z = h_ref[...] @ w0[...] + b0[...]
        z = z @ w1[...] + b1[...]
        z = jnp.maximum(z, 0.0)
        z = z @ w2[...] + b2[...]
        colmask = jax.lax.broadcasted_iota(jnp.int32, (NG, 128), 1) < 10
        zm = jnp.where(colmask, z, -jnp.inf)
        mx = jnp.max(zm, axis=1, keepdims=True)
        lse = mx + jnp.log(jnp.sum(jnp.where(colmask, jnp.exp(zm - mx), 0.0),
                                   axis=1, keepdims=True))
        o_ref[...] = z - lse

    def wspec(s):
        return pl.BlockSpec(s, lambda: tuple(0 for _ in s))

    out = pl.pallas_call(
        body,
        in_specs=[wspec((NG, 3 * D)), wspec((3 * D, 128)), wspec((1, 128)),
                  wspec((128, 256)), wspec((1, 256)),
                  wspec((256, 128)), wspec((1, 128))],
        out_specs=wspec((NG, 128)),
        out_shape=jax.ShapeDtypeStruct((NG, 128), jnp.float32),
    )(H, cw[0], cb[0].reshape(1, 128), cw[1], cb[1].reshape(1, 256), W2p, b2p)
    return out[:, :10]


def _split_agg(A, n):
    s = A[0] + A[1]
    return (s[:n, :D], s[:n, D:D + 3], s[:n, D + 3:D + 4], s[:n, D + 4:D + 5])


def _dcalc(x, xsq, Sx, Ssq, cnt, deg):
    return (xsq * cnt + Ssq - 2.0 * jnp.sum(x * Sx, axis=1, keepdims=True)) / deg


def kernel(h_1, h_2, h_3, x_1, b_1, b_2, batch1, batch2, batch3, params):
    p = params
    e1_rs, e1_rd, e1_cs, e1_cd = _extract_edges(b_1, N1, N2)
    e2_rs, e2_rd, e2_cs, e2_cd = _extract_edges(b_2, N2, N3)
    z1 = jnp.zeros((NP1, F), jnp.float32)
    z2 = jnp.zeros((NP2, F), jnp.float32)
    z3 = jnp.zeros((NP3, F), jnp.float32)

    h1 = _affine(h_1, p['init_W'][0], p['init_b'][0])
    h2 = _affine(h_2, p['init_W'][1], p['init_b'][1])
    h3 = _affine(h_3, p['init_W'][2], p['init_b'][2])
    x1 = x_1

    for i in range(DEPTH):
        lp = p['layers'][i]
        T1 = _table(h1, x1, NP1)
        A2 = _sc_pass(T1, e1_cs, e1_cd, z2, NP2)
        S2h, S2x, S2sq, c2 = _split_agg(A2, N2)
        x2 = S2x / 2.0
        T2 = _table(h2, x2, NP2)
        A3 = _sc_pass(T2, e2_cs, e2_cd, z3, NP3)
        S3h, S3x, S3sq, c3 = _split_agg(A3, N3)
        x3 = S3x / 3.0
        A1 = _sc_pass(T2, e1_rs, e1_rd, z1, NP1)
        S1h, S1x, S1sq, c1 = _split_agg(A1, N1)
        T3 = _table(h3, x3, NP3)
        A2u = _sc_pass(T3, e2_rs, e2_rd, z2, NP2)
        S2uh, S2ux, S2usq, c2u = _split_agg(A2u, N2)

        deg1 = jnp.maximum(c1, 1.0)
        xsq1 = jnp.sum(x1 * x1, axis=1, keepdims=True)
        d1 = _dcalc(x1, xsq1, S1x, S1sq, c1, deg1)
        l1 = lp[0]
        h1n, coef = _lvl1_call(h1, S1h, deg1, d1,
                               (l1['up_W1'], l1['up_b1'], l1['up_W2'], l1['up_b2'],
                                l1['h_W1'], l1['h_b1'], l1['h_W2'], l1['h_b2'],
                                l1['x_W'], l1['x_b']))
        rel = (x1 * c1 - S1x) / deg1
        x1n = x1 + rel * coef

        deg2u = jnp.maximum(c2u, 1.0)
        deg2d = jnp.maximum(c2, 1.0)
        xsq2 = jnp.sum(x2 * x2, axis=1, keepdims=True)
        d2u = _dcalc(x2, xsq2, S2ux, S2usq, c2u, deg2u)
        d2d = _dcalc(x2, xsq2, S2x, S2sq, c2, deg2d)
        l2 = lp[1]
        h2n = _lvl2_call(h2, S2uh, deg2u, d2u, S2h, deg2d, d2d,
                         (l2['up_W1'], l2['up_b1'], l2['up_W2'], l2['up_b2'],
                          l2['dn_W1'], l2['dn_b1'], l2['dn_W2'], l2['dn_b2'],
                          l2['h_W1'], l2['h_b1'], l2['h_W2'], l2['h_b2']))

        deg3 = jnp.maximum(c3, 1.0)
        xsq3 = jnp.sum(x3 * x3, axis=1, keepdims=True)
        d3 = _dcalc(x3, xsq3, S3x, S3sq, c3, deg3)
        l3 = lp[2]
        h3n, _ = _lvl1_call(h3, S3h, deg3, d3,
                            (l3['dn_W1'], l3['dn_b1'], l3['dn_W2'], l3['dn_b2'],
                             l3['h_W1'], l3['h_b1'], l3['h_W2'], l3['h_b2'],
                             jnp.zeros((D, 1), jnp.float32),
                             jnp.zeros((1,), jnp.float32)))
        h1, h2, h3, x1 = h1n, h2n, h3n, x1n

    h1b = _segmax(h1, batch1)
    h2b = _segmax(h2, batch2)
    h3b = _segmax(h3, batch3)
    H = jnp.concatenate([h1b, h2b, h3b], axis=-1)
    return _classifier(H, p['cls_W'], p['cls_b'])
